# Initial kernel scaffold; baseline (speedup 1.0000x reference)
import functools
import jax
import jax.numpy as jnp
import numpy as np
from jax import lax
from jax.experimental import pallas as pl
from jax.experimental.pallas import tpu as pltpu
from jax.experimental.pallas import tpu_sc as plsc

N_NODES = 10000
N_EDGES = 320000
D = 128
H = 4
D_H = 32

NP = 10240
NW = 32
CHUNK = 128
CHUNKS_PER_W = 79
EP = NW * CHUNKS_PER_W * CHUNK
ROWS_PER_TILE = NP // 16
NBLK = 256
LEAK = 0.2


def _tc1_body(x_ref, w_ref, asrc_ref, adst_ref, wx_ref, ssrc_ref, sdst_ref):
    b = pl.program_id(0)
    wx = jnp.dot(x_ref[...], w_ref[...], preferred_element_type=jnp.float32)
    ssrc = jnp.dot(wx, asrc_ref[...], preferred_element_type=jnp.float32)
    sdst = jnp.dot(wx, adst_ref[...], preferred_element_type=jnp.float32)
    rows = b * NBLK + lax.broadcasted_iota(jnp.int32, (NBLK, 1), 0)
    ssrc = jnp.where(rows < N_NODES, ssrc, jnp.float32(-1e30))
    wx_ref[...] = wx
    ssrc_ref[...] = ssrc
    sdst_ref[...] = sdst


def _tc2_body(a0_ref, a1_ref, z0_ref, z1_ref, m_ref, wot_ref, bo_ref, out_ref):
    agg = a0_ref[...] + a1_ref[...]
    z = z0_ref[...] + z1_ref[...]
    d = jnp.dot(z, m_ref[...], preferred_element_type=jnp.float32) + 1e-16
    o = jnp.dot(agg / d, wot_ref[...], preferred_element_type=jnp.float32)
    o = o + bo_ref[...]
    out_ref[...] = jnp.where(o > 0, o, jnp.expm1(o))


def _sc_body(ssrc_h, sdst_h, wx_h, src_h, dst_h, za_h, zz_h,
             agg_out, z_out,
             src_v, dst_v, gs, gd, gwx, wbuf, agg_s, z_s,
             sem0, sem1, sem2):
    c = lax.axis_index("c")
    s = lax.axis_index("s")
    t = c * 16 + s

    stripe = pl.ds(s * ROWS_PER_TILE, ROWS_PER_TILE)
    pltpu.sync_copy(za_h, agg_s.at[stripe])
    pltpu.sync_copy(zz_h, z_s.at[stripe])
    pltpu.sync_copy(src_h.at[t], src_v)
    pltpu.sync_copy(dst_h.at[t], dst_v)
    plsc.subcore_barrier()

    lmask = lax.iota(jnp.int32, 16) < 4

    @pl.loop(0, CHUNKS_PER_W)
    def _chunk(cc):
        si = src_v.at[cc]
        di = dst_v.at[cc]
        cp0 = pltpu.async_copy(ssrc_h.at[si], gs, sem0)
        cp1 = pltpu.async_copy(sdst_h.at[di], gd, sem1)
        cp2 = pltpu.async_copy(wx_h.at[si], gwx, sem2)
        cp0.wait()
        cp1.wait()
        cp2.wait()

        @pl.loop(0, CHUNK)
        def _edge(i):
            es = gs[i] + gd[i]
            e = jnp.where(es > 0, es, es * LEAK)
            w = jnp.where(lmask, jnp.exp(e), jnp.float32(0.0))
            wbuf[i] = w
            for h in range(H):
                sc = jnp.sum(w * (lax.iota(jnp.int32, 16) == h).astype(jnp.float32))
                bc = jnp.full((16,), sc, jnp.float32)
                for q in range(2):
                    sl = pl.ds(h * 32 + q * 16, 16)
                    gwx[i, sl] = gwx[i, sl] * bc

        pltpu.sync_copy(wbuf, z_s.at[di], add=True)
        pltpu.sync_copy(gwx, agg_s.at[di], add=True)

    plsc.subcore_barrier()
    pltpu.sync_copy(agg_s.at[stripe], agg_out.at[c, stripe])
    pltpu.sync_copy(z_s.at[stripe], z_out.at[c, stripe])


def kernel(x, edge_index, W, attn_vec, Wo, bo):
    f32 = jnp.float32

    asrc = jnp.zeros((D, 16), f32)
    adst = jnp.zeros((D, 16), f32)
    for h in range(H):
        asrc = asrc.at[h * D_H:(h + 1) * D_H, h].set(attn_vec[h, :D_H])
        adst = adst.at[h * D_H:(h + 1) * D_H, h].set(attn_vec[h, D_H:])
    m16 = np.zeros((16, D), np.float32)
    for l in range(D):
        m16[l // D_H, l] = 1.0
    m16 = jnp.asarray(m16)

    xp = jnp.concatenate([x, jnp.zeros((NP - N_NODES, D), f32)])
    srcp = jnp.concatenate(
        [edge_index[0], jnp.full((EP - N_EDGES,), N_NODES, jnp.int32)]
    ).reshape(NW, CHUNKS_PER_W, CHUNK)
    dstp = jnp.concatenate(
        [edge_index[1], jnp.full((EP - N_EDGES,), N_NODES, jnp.int32)]
    ).reshape(NW, CHUNKS_PER_W, CHUNK)

    grid1 = (NP // NBLK,)
    wxp, ssrcp, sdstp = pl.pallas_call(
        _tc1_body,
        grid=grid1,
        in_specs=[
            pl.BlockSpec((NBLK, D), lambda b: (b, 0)),
            pl.BlockSpec((D, D), lambda b: (0, 0)),
            pl.BlockSpec((D, 16), lambda b: (0, 0)),
            pl.BlockSpec((D, 16), lambda b: (0, 0)),
        ],
        out_specs=[
            pl.BlockSpec((NBLK, D), lambda b: (b, 0)),
            pl.BlockSpec((NBLK, 16), lambda b: (b, 0)),
            pl.BlockSpec((NBLK, 16), lambda b: (b, 0)),
        ],
        out_shape=[
            jax.ShapeDtypeStruct((NP, D), f32),
            jax.ShapeDtypeStruct((NP, 16), f32),
            jax.ShapeDtypeStruct((NP, 16), f32),
        ],
    )(xp, W, asrc, adst)

    mesh = plsc.VectorSubcoreMesh(core_axis_name="c", subcore_axis_name="s")
    sc_kernel = pl.kernel(
        _sc_body,
        out_type=[
            jax.ShapeDtypeStruct((2, NP, D), f32),
            jax.ShapeDtypeStruct((2, NP, 16), f32),
        ],
        mesh=mesh,
        scratch_types=[
            pltpu.VMEM((CHUNKS_PER_W, CHUNK), jnp.int32),
            pltpu.VMEM((CHUNKS_PER_W, CHUNK), jnp.int32),
            pltpu.VMEM((CHUNK, 16), f32),
            pltpu.VMEM((CHUNK, 16), f32),
            pltpu.VMEM((CHUNK, D), f32),
            pltpu.VMEM((CHUNK, 16), f32),
            pltpu.VMEM_SHARED((NP, D), f32),
            pltpu.VMEM_SHARED((NP, 16), f32),
            pltpu.SemaphoreType.DMA,
            pltpu.SemaphoreType.DMA,
            pltpu.SemaphoreType.DMA,
        ],
    )
    za = jnp.zeros((ROWS_PER_TILE, D), f32)
    zz = jnp.zeros((ROWS_PER_TILE, 16), f32)
    agg, z = sc_kernel(ssrcp, sdstp, wxp, srcp, dstp, za, zz)

    grid3 = (NP // NBLK,)
    out = pl.pallas_call(
        _tc2_body,
        grid=grid3,
        in_specs=[
            pl.BlockSpec((NBLK, D), lambda b: (b, 0)),
            pl.BlockSpec((NBLK, D), lambda b: (b, 0)),
            pl.BlockSpec((NBLK, 16), lambda b: (b, 0)),
            pl.BlockSpec((NBLK, 16), lambda b: (b, 0)),
            pl.BlockSpec((16, D), lambda b: (0, 0)),
            pl.BlockSpec((D, D), lambda b: (0, 0)),
            pl.BlockSpec((1, D), lambda b: (0, 0)),
        ],
        out_specs=pl.BlockSpec((NBLK, D), lambda b: (b, 0)),
        out_shape=jax.ShapeDtypeStruct((NP, D), f32),
    )(agg[0], agg[1], z[0], z[1], m16, Wo.T, bo.reshape(1, D))

    return out[:N_NODES]


# speedup vs baseline: 37.9856x; 37.9856x over previous
import dataclasses
import functools
import jax
import jax.numpy as jnp
import numpy as np
from jax import lax
from jax.experimental import pallas as pl
from jax.experimental.pallas import tpu as pltpu
from jax.experimental.pallas import tpu_sc as plsc

N_NODES = 10000
N_EDGES = 320000
D = 128
DH2 = 64
H = 4
D_H = 32

NP = 10240
NS = 16
CHUNK = 128
CHUNKS_PER_S = 158
EP = NS * CHUNKS_PER_S * CHUNK
ROWS_PER_TILE = NP // NS
ZROWS = 40
NBLK = 256
LEAK = 0.2


def _tc1_body(x_ref, w_ref, asrc_ref, adst_ref, wx_ref, ssrc_ref, sdst_ref):
    b = pl.program_id(0)
    wx = jnp.dot(x_ref[...], w_ref[...], preferred_element_type=jnp.float32)
    ssrc = jnp.dot(wx, asrc_ref[...], preferred_element_type=jnp.float32)
    sdst = jnp.dot(wx, adst_ref[...], preferred_element_type=jnp.float32)
    rows = b * NBLK + lax.broadcasted_iota(jnp.int32, (NBLK, 1), 0)
    ssrc = jnp.where(rows < N_NODES, ssrc, jnp.float32(-1e30))
    wx_ref[...] = wx
    ssrc_ref[...] = ssrc
    sdst_ref[...] = sdst


def _tc2_body(a0_ref, a1_ref, z0_ref, z1_ref, m_ref, wot_ref, bo_ref, out_ref):
    agg = jnp.concatenate([a0_ref[...], a1_ref[...]], axis=1)
    z = z0_ref[...] + z1_ref[...]
    d = jnp.dot(z, m_ref[...], preferred_element_type=jnp.float32) + 1e-16
    o = jnp.dot(agg / d, wot_ref[...], preferred_element_type=jnp.float32)
    o = o + bo_ref[...]
    out_ref[...] = jnp.where(o > 0, o, jnp.exp(o) - 1.0)


def _sc_body(ssrc_h, sdst_h, wxs_h, src_h, dst_h, za_h, zz_h,
             agg_out, z_out,
             src_v, dst_v, gs, gd, gwx, wbuf, agg_s, z_s,
             sem0, sem1, sem2):
    c = lax.axis_index("c")
    s = lax.axis_index("s")

    @pl.loop(0, ROWS_PER_TILE // ZROWS)
    def _zero(r):
        base = s * ROWS_PER_TILE + r * ZROWS
        pltpu.sync_copy(za_h, agg_s.at[pl.ds(base, ZROWS)])
        pltpu.sync_copy(zz_h, z_s.at[pl.ds(base, ZROWS)])

    pltpu.sync_copy(src_h.at[s], src_v)
    pltpu.sync_copy(dst_h.at[s], dst_v)
    plsc.subcore_barrier()

    lane = lax.iota(jnp.int32, 16)
    h0 = c * 2
    hmask = (lane >= h0) & (lane < h0 + 2)
    oh0 = (lane == h0).astype(jnp.float32)
    oh1 = (lane == h0 + 1).astype(jnp.float32)

    @pl.loop(0, CHUNKS_PER_S)
    def _chunk(cc):
        si = src_v.at[cc]
        di = dst_v.at[cc]
        cp0 = pltpu.async_copy(ssrc_h.at[si], gs, sem0)
        cp1 = pltpu.async_copy(sdst_h.at[di], gd, sem1)
        cp2 = pltpu.async_copy(wxs_h.at[c].at[si], gwx, sem2)
        cp0.wait()
        cp1.wait()
        cp2.wait()

        @pl.loop(0, CHUNK)
        def _edge(i):
            es = gs[i] + gd[i]
            e = jnp.where(es > 0, es, es * LEAK)
            w = jnp.where(hmask, jnp.exp(e), jnp.float32(0.0))
            wbuf[i] = w
            for hh in range(2):
                oh = oh0 if hh == 0 else oh1
                sc = jnp.sum(w * oh)
                bc = jnp.full((16,), sc, jnp.float32)
                for q in range(2):
                    sl = pl.ds(hh * 32 + q * 16, 16)
                    gwx[i, sl] = gwx[i, sl] * bc

        pltpu.sync_copy(wbuf, z_s.at[di], add=True)
        pltpu.sync_copy(gwx, agg_s.at[di], add=True)

    plsc.subcore_barrier()
    stripe = pl.ds(s * ROWS_PER_TILE, ROWS_PER_TILE)
    pltpu.sync_copy(agg_s.at[stripe], agg_out.at[c, stripe])
    pltpu.sync_copy(z_s.at[stripe], z_out.at[c, stripe])


def kernel(x, edge_index, W, attn_vec, Wo, bo):
    f32 = jnp.float32

    asrc = jnp.zeros((D, 16), f32)
    adst = jnp.zeros((D, 16), f32)
    for h in range(H):
        asrc = asrc.at[h * D_H:(h + 1) * D_H, h].set(attn_vec[h, :D_H])
        adst = adst.at[h * D_H:(h + 1) * D_H, h].set(attn_vec[h, D_H:])
    m16 = np.zeros((16, D), np.float32)
    for l in range(D):
        m16[l // D_H, l] = 1.0
    m16 = jnp.asarray(m16)

    xp = jnp.concatenate([x, jnp.zeros((NP - N_NODES, D), f32)])
    srcp = jnp.concatenate(
        [edge_index[0], jnp.full((EP - N_EDGES,), N_NODES, jnp.int32)]
    ).reshape(NS, CHUNKS_PER_S, CHUNK)
    dstp = jnp.concatenate(
        [edge_index[1], jnp.full((EP - N_EDGES,), N_NODES, jnp.int32)]
    ).reshape(NS, CHUNKS_PER_S, CHUNK)

    grid1 = (NP // NBLK,)
    wxp, ssrcp, sdstp = pl.pallas_call(
        _tc1_body,
        grid=grid1,
        in_specs=[
            pl.BlockSpec((NBLK, D), lambda b: (b, 0)),
            pl.BlockSpec((D, D), lambda b: (0, 0)),
            pl.BlockSpec((D, 16), lambda b: (0, 0)),
            pl.BlockSpec((D, 16), lambda b: (0, 0)),
        ],
        out_specs=[
            pl.BlockSpec((NBLK, D), lambda b: (b, 0)),
            pl.BlockSpec((NBLK, 16), lambda b: (b, 0)),
            pl.BlockSpec((NBLK, 16), lambda b: (b, 0)),
        ],
        out_shape=[
            jax.ShapeDtypeStruct((NP, D), f32),
            jax.ShapeDtypeStruct((NP, 16), f32),
            jax.ShapeDtypeStruct((NP, 16), f32),
        ],
    )(xp, W, asrc, adst)

    wxsplit = jnp.stack([wxp[:, :DH2], wxp[:, DH2:]])

    mesh = plsc.VectorSubcoreMesh(core_axis_name="c", subcore_axis_name="s")
    cp = pltpu.CompilerParams(
        needs_layout_passes=False, use_tc_tiling_on_sc=False
    )
    sc_kernel = pl.kernel(
        _sc_body,
        compiler_params=cp,
        out_type=[
            jax.ShapeDtypeStruct((2, NP, DH2), f32),
            jax.ShapeDtypeStruct((2, NP, 16), f32),
        ],
        mesh=mesh,
        scratch_types=[
            pltpu.VMEM((CHUNKS_PER_S, CHUNK), jnp.int32),
            pltpu.VMEM((CHUNKS_PER_S, CHUNK), jnp.int32),
            pltpu.VMEM((CHUNK, 16), f32),
            pltpu.VMEM((CHUNK, 16), f32),
            pltpu.VMEM((CHUNK, DH2), f32),
            pltpu.VMEM((CHUNK, 16), f32),
            pltpu.VMEM_SHARED((NP, DH2), f32),
            pltpu.VMEM_SHARED((NP, 16), f32),
            pltpu.SemaphoreType.DMA,
            pltpu.SemaphoreType.DMA,
            pltpu.SemaphoreType.DMA,
        ],
    )
    za = jnp.zeros((ZROWS, DH2), f32)
    zz = jnp.zeros((ZROWS, 16), f32)
    agg, z = sc_kernel(ssrcp, sdstp, wxsplit, srcp, dstp, za, zz)

    grid3 = (NP // NBLK,)
    out = pl.pallas_call(
        _tc2_body,
        grid=grid3,
        in_specs=[
            pl.BlockSpec((NBLK, DH2), lambda b: (b, 0)),
            pl.BlockSpec((NBLK, DH2), lambda b: (b, 0)),
            pl.BlockSpec((NBLK, 16), lambda b: (b, 0)),
            pl.BlockSpec((NBLK, 16), lambda b: (b, 0)),
            pl.BlockSpec((16, D), lambda b: (0, 0)),
            pl.BlockSpec((D, D), lambda b: (0, 0)),
            pl.BlockSpec((1, D), lambda b: (0, 0)),
        ],
        out_specs=pl.BlockSpec((NBLK, D), lambda b: (b, 0)),
        out_shape=jax.ShapeDtypeStruct((NP, D), f32),
    )(agg[0], agg[1], z[0], z[1], m16, Wo.T, bo.reshape(1, D))

    return out[:N_NODES]


# speedup vs baseline: 42.9073x; 1.1296x over previous
import dataclasses
import functools
import jax
import jax.numpy as jnp
import numpy as np
from jax import lax
from jax.experimental import pallas as pl
from jax.experimental.pallas import tpu as pltpu
from jax.experimental.pallas import tpu_sc as plsc

N_NODES = 10000
N_EDGES = 320000
D = 128
DH2 = 64
H = 4
D_H = 32

NP = 10240
NS = 16
CHUNK = 128
CHUNKS_PER_S = 158
EP = NS * CHUNKS_PER_S * CHUNK
ROWS_PER_TILE = NP // NS
ZROWS = 40
NBLK = 256
LEAK = 0.2


def _tc1_body(x_ref, w_ref, asrc_ref, adst_ref, wx_ref, ssrc_ref, sdst_ref):
    b = pl.program_id(0)
    wx = jnp.dot(x_ref[...], w_ref[...], preferred_element_type=jnp.float32)
    ssrc = jnp.dot(wx, asrc_ref[...], preferred_element_type=jnp.float32)
    sdst = jnp.dot(wx, adst_ref[...], preferred_element_type=jnp.float32)
    rows = b * NBLK + lax.broadcasted_iota(jnp.int32, (NBLK, 1), 0)
    ssrc = jnp.where(rows < N_NODES, ssrc, jnp.float32(-1e30))
    wx_ref[...] = wx
    ssrc_ref[...] = ssrc
    sdst_ref[...] = sdst


def _tc2_body(a0_ref, a1_ref, z0_ref, z1_ref, m_ref, wot_ref, bo_ref, out_ref):
    agg = jnp.concatenate([a0_ref[...], a1_ref[...]], axis=1)
    z = z0_ref[...] + z1_ref[...]
    d = jnp.dot(z, m_ref[...], preferred_element_type=jnp.float32) + 1e-16
    o = jnp.dot(agg / d, wot_ref[...], preferred_element_type=jnp.float32)
    o = o + bo_ref[...]
    out_ref[...] = jnp.where(o > 0, o, jnp.exp(o) - 1.0)


def _sc_body(ssrc_h, sdst_h, wxs_h, src_h, dst_h, za_h, zz_h,
             agg_out, z_out,
             src_v, dst_v, gs, gd, gwx, wbuf, agg_s, z_s,
             sem0, sem1, sem2):
    c = lax.axis_index("c")
    s = lax.axis_index("s")

    @pl.loop(0, ROWS_PER_TILE // ZROWS)
    def _zero(r):
        base = s * ROWS_PER_TILE + r * ZROWS
        pltpu.sync_copy(za_h, agg_s.at[pl.ds(base, ZROWS)])
        pltpu.sync_copy(zz_h, z_s.at[pl.ds(base, ZROWS)])

    pltpu.sync_copy(src_h.at[s], src_v)
    pltpu.sync_copy(dst_h.at[s], dst_v)
    plsc.subcore_barrier()

    lane = lax.iota(jnp.int32, 16)
    h0 = c * 2
    hmask = (lane >= h0) & (lane < h0 + 2)
    bcidx = [jnp.full((16,), h0 + hh, jnp.int32) for hh in range(2)]

    def _splat(v, idx):
        return lax.gather(
            v, idx[:, None],
            lax.GatherDimensionNumbers(
                offset_dims=(), collapsed_slice_dims=(0,),
                start_index_map=(0,)),
            (1,), mode=lax.GatherScatterMode.PROMISE_IN_BOUNDS)

    @pl.loop(0, CHUNKS_PER_S)
    def _chunk(cc):
        si = src_v.at[cc]
        di = dst_v.at[cc]
        cp0 = pltpu.async_copy(ssrc_h.at[si], gs, sem0)
        cp1 = pltpu.async_copy(sdst_h.at[di], gd, sem1)
        cp2 = pltpu.async_copy(wxs_h.at[c].at[si], gwx, sem2)
        cp0.wait()
        cp1.wait()
        cp2.wait()

        @pl.loop(0, CHUNK, step=4)
        def _edge(ii):
            for u in range(4):
                i = ii + u
                es = gs[i] + gd[i]
                e = jnp.where(es > 0, es, es * LEAK)
                w = jnp.where(hmask, jnp.exp(e), jnp.float32(0.0))
                wbuf[i] = w
                for hh in range(2):
                    bc = _splat(w, bcidx[hh])
                    for q in range(2):
                        sl = pl.ds(hh * 32 + q * 16, 16)
                        gwx[i, sl] = gwx[i, sl] * bc

        pltpu.sync_copy(wbuf, z_s.at[di], add=True)
        pltpu.sync_copy(gwx, agg_s.at[di], add=True)

    plsc.subcore_barrier()
    stripe = pl.ds(s * ROWS_PER_TILE, ROWS_PER_TILE)
    pltpu.sync_copy(agg_s.at[stripe], agg_out.at[c, stripe])
    pltpu.sync_copy(z_s.at[stripe], z_out.at[c, stripe])


def kernel(x, edge_index, W, attn_vec, Wo, bo):
    f32 = jnp.float32

    asrc = jnp.zeros((D, 16), f32)
    adst = jnp.zeros((D, 16), f32)
    for h in range(H):
        asrc = asrc.at[h * D_H:(h + 1) * D_H, h].set(attn_vec[h, :D_H])
        adst = adst.at[h * D_H:(h + 1) * D_H, h].set(attn_vec[h, D_H:])
    m16 = np.zeros((16, D), np.float32)
    for l in range(D):
        m16[l // D_H, l] = 1.0
    m16 = jnp.asarray(m16)

    xp = jnp.concatenate([x, jnp.zeros((NP - N_NODES, D), f32)])
    srcp = jnp.concatenate(
        [edge_index[0], jnp.full((EP - N_EDGES,), N_NODES, jnp.int32)]
    ).reshape(NS, CHUNKS_PER_S, CHUNK)
    dstp = jnp.concatenate(
        [edge_index[1], jnp.full((EP - N_EDGES,), N_NODES, jnp.int32)]
    ).reshape(NS, CHUNKS_PER_S, CHUNK)

    grid1 = (NP // NBLK,)
    wxp, ssrcp, sdstp = pl.pallas_call(
        _tc1_body,
        grid=grid1,
        in_specs=[
            pl.BlockSpec((NBLK, D), lambda b: (b, 0)),
            pl.BlockSpec((D, D), lambda b: (0, 0)),
            pl.BlockSpec((D, 16), lambda b: (0, 0)),
            pl.BlockSpec((D, 16), lambda b: (0, 0)),
        ],
        out_specs=[
            pl.BlockSpec((NBLK, D), lambda b: (b, 0)),
            pl.BlockSpec((NBLK, 16), lambda b: (b, 0)),
            pl.BlockSpec((NBLK, 16), lambda b: (b, 0)),
        ],
        out_shape=[
            jax.ShapeDtypeStruct((NP, D), f32),
            jax.ShapeDtypeStruct((NP, 16), f32),
            jax.ShapeDtypeStruct((NP, 16), f32),
        ],
    )(xp, W, asrc, adst)

    wxsplit = jnp.stack([wxp[:, :DH2], wxp[:, DH2:]])

    mesh = plsc.VectorSubcoreMesh(core_axis_name="c", subcore_axis_name="s")
    cp = pltpu.CompilerParams(
        needs_layout_passes=False, use_tc_tiling_on_sc=False
    )
    sc_kernel = pl.kernel(
        _sc_body,
        compiler_params=cp,
        out_type=[
            jax.ShapeDtypeStruct((2, NP, DH2), f32),
            jax.ShapeDtypeStruct((2, NP, 16), f32),
        ],
        mesh=mesh,
        scratch_types=[
            pltpu.VMEM((CHUNKS_PER_S, CHUNK), jnp.int32),
            pltpu.VMEM((CHUNKS_PER_S, CHUNK), jnp.int32),
            pltpu.VMEM((CHUNK, 16), f32),
            pltpu.VMEM((CHUNK, 16), f32),
            pltpu.VMEM((CHUNK, DH2), f32),
            pltpu.VMEM((CHUNK, 16), f32),
            pltpu.VMEM_SHARED((NP, DH2), f32),
            pltpu.VMEM_SHARED((NP, 16), f32),
            pltpu.SemaphoreType.DMA,
            pltpu.SemaphoreType.DMA,
            pltpu.SemaphoreType.DMA,
        ],
    )
    za = jnp.zeros((ZROWS, DH2), f32)
    zz = jnp.zeros((ZROWS, 16), f32)
    agg, z = sc_kernel(ssrcp, sdstp, wxsplit, srcp, dstp, za, zz)

    grid3 = (NP // NBLK,)
    out = pl.pallas_call(
        _tc2_body,
        grid=grid3,
        in_specs=[
            pl.BlockSpec((NBLK, DH2), lambda b: (b, 0)),
            pl.BlockSpec((NBLK, DH2), lambda b: (b, 0)),
            pl.BlockSpec((NBLK, 16), lambda b: (b, 0)),
            pl.BlockSpec((NBLK, 16), lambda b: (b, 0)),
            pl.BlockSpec((16, D), lambda b: (0, 0)),
            pl.BlockSpec((D, D), lambda b: (0, 0)),
            pl.BlockSpec((1, D), lambda b: (0, 0)),
        ],
        out_specs=pl.BlockSpec((NBLK, D), lambda b: (b, 0)),
        out_shape=jax.ShapeDtypeStruct((NP, D), f32),
    )(agg[0], agg[1], z[0], z[1], m16, Wo.T, bo.reshape(1, D))

    return out[:N_NODES]
